# baseline (speedup 1.0000x reference)
import functools
from typing import Any

import jax
import jax.numpy as jnp
from jax import lax
from jax.experimental import pallas as pl
from jax.experimental.pallas import tpu as pltpu
from jax.experimental.pallas import tpu_sc as plsc

HID = 128
K = 32
N_PROT = 4800
N_LIG = 1200
N = N_PROT + N_LIG
NPAD = 6144
XW = 16
TD = 256
BLK = 256
NBLK = NPAD // BLK
E = NPAD * K
CHUNK = 128


def _silu(v):
    return v * jax.nn.sigmoid(v)


def _embed_body(v_ref, w_ref, b_ref, o_ref):
    o_ref[...] = jnp.dot(v_ref[...], w_ref[...],
                         preferred_element_type=jnp.float32) + b_ref[...]


def _embed(v, w, b):
    return pl.pallas_call(
        _embed_body,
        out_shape=jax.ShapeDtypeStruct((v.shape[0], HID), jnp.float32),
    )(v, w, b)


def _knn_body(xrow_ref, xt_ref, brow_ref, bcol_ref, src_ref, d2_ref):
    i = pl.program_id(0)
    xi = xrow_ref[...]
    bi = brow_ref[...]
    bj = bcol_ref[...]
    d2 = jnp.zeros((BLK, NPAD), jnp.float32)
    for c in range(3):
        diff = xi[:, c:c + 1] - xt_ref[c:c + 1, :]
        d2 = d2 + diff * diff
    rowid = i * BLK + lax.broadcasted_iota(jnp.int32, (BLK, 1), 0)
    colid = lax.broadcasted_iota(jnp.int32, (BLK, NPAD), 1)
    same = (bi == bj) & (rowid != colid)
    d2m = jnp.where(same, d2, jnp.float32(1e30))
    for k in range(K):
        idx = jnp.argmin(d2m, axis=1).astype(jnp.int32)[:, None]
        hit = colid == idx
        val = jnp.min(jnp.where(hit, d2, jnp.float32(3e30)),
                      axis=1, keepdims=True)
        src_ref[:, k:k + 1] = idx
        d2_ref[:, k:k + 1] = val
        d2m = jnp.where(hit, jnp.float32(2e30), d2m)


def _knn(xrow, xt, brow, bcol):
    return pl.pallas_call(
        _knn_body,
        grid=(NBLK,),
        in_specs=[
            pl.BlockSpec((BLK, XW), lambda i: (i, 0)),
            pl.BlockSpec((8, NPAD), lambda i: (0, 0)),
            pl.BlockSpec((BLK, 1), lambda i: (i, 0)),
            pl.BlockSpec((1, NPAD), lambda i: (0, 0)),
        ],
        out_specs=[
            pl.BlockSpec((BLK, K), lambda i: (i, 0)),
            pl.BlockSpec((BLK, K), lambda i: (i, 0)),
        ],
        out_shape=[
            jax.ShapeDtypeStruct((NPAD, K), jnp.int32),
            jax.ShapeDtypeStruct((NPAD, K), jnp.float32),
        ],
    )(xrow, xt, brow, bcol)


def _prep_body(h_ref, x_ref, we1a_ref, we1b_ref, wh1a_ref, be1_ref, bh1_ref,
               a_ref, c_ref, t_ref):
    h = h_ref[...]
    a_ref[...] = jnp.dot(h, we1a_ref[...],
                         preferred_element_type=jnp.float32) + be1_ref[...]
    c_ref[...] = jnp.dot(h, wh1a_ref[...],
                         preferred_element_type=jnp.float32) + bh1_ref[...]
    b = jnp.dot(h, we1b_ref[...], preferred_element_type=jnp.float32)
    t_ref[...] = jnp.concatenate(
        [b, x_ref[...], jnp.zeros((BLK, TD - HID - XW), jnp.float32)], axis=1)


def _prep(h, x, we1a, we1b, wh1a, be1, bh1):
    return pl.pallas_call(
        _prep_body,
        grid=(NBLK,),
        in_specs=[
            pl.BlockSpec((BLK, HID), lambda i: (i, 0)),
            pl.BlockSpec((BLK, XW), lambda i: (i, 0)),
            pl.BlockSpec((HID, HID), lambda i: (0, 0)),
            pl.BlockSpec((HID, HID), lambda i: (0, 0)),
            pl.BlockSpec((HID, HID), lambda i: (0, 0)),
            pl.BlockSpec((1, HID), lambda i: (0, 0)),
            pl.BlockSpec((1, HID), lambda i: (0, 0)),
        ],
        out_specs=[
            pl.BlockSpec((BLK, HID), lambda i: (i, 0)),
            pl.BlockSpec((BLK, HID), lambda i: (i, 0)),
            pl.BlockSpec((BLK, TD), lambda i: (i, 0)),
        ],
        out_shape=[
            jax.ShapeDtypeStruct((NPAD, HID), jnp.float32),
            jax.ShapeDtypeStruct((NPAD, HID), jnp.float32),
            jax.ShapeDtypeStruct((NPAD, TD), jnp.float32),
        ],
    )(h, x, we1a, we1b, wh1a, be1, bh1)


_PER_W = E // 32
_NCHUNK = _PER_W // CHUNK


def _sc_gather(table, idx):
    mesh = plsc.VectorSubcoreMesh(core_axis_name="c", subcore_axis_name="s")

    @functools.partial(
        pl.kernel,
        mesh=mesh,
        out_type=jax.ShapeDtypeStruct((E, TD), jnp.float32),
        scratch_types=[
            pltpu.VMEM((_PER_W,), jnp.int32),
            pltpu.VMEM((CHUNK, TD), jnp.float32),
            pltpu.VMEM((CHUNK, TD), jnp.float32),
            pltpu.SemaphoreType.DMA,
            pltpu.SemaphoreType.DMA,
        ],
    )
    def k(table_hbm, idx_hbm, out_hbm, idx_v, rows0, rows1, sem0, sem1):
        wid = lax.axis_index("s") * 2 + lax.axis_index("c")
        base0 = wid * _PER_W
        pltpu.sync_copy(idx_hbm.at[pl.ds(base0, _PER_W)], idx_v)
        bufs = [(rows0, sem0), (rows1, sem1)]

        def fire(c):
            r, s = bufs[c % 2]
            return pltpu.async_copy(
                table_hbm.at[idx_v.at[pl.ds(c * CHUNK, CHUNK)]], r, s)

        cp = fire(0)
        for c in range(_NCHUNK):
            nxt = fire(c + 1) if c + 1 < _NCHUNK else None
            cp.wait()
            r, _ = bufs[c % 2]
            pltpu.sync_copy(r, out_hbm.at[pl.ds(base0 + c * CHUNK, CHUNK)])
            cp = nxt

    return k(table, idx)


def _edge_body(a_ref, c_ref, h_ref, x_ref, d2_ref, g_ref, mask_ref,
               wd_ref, we2_ref, be2_ref, winf_ref, binf_ref,
               wx1_ref, bx1_ref, wx2_ref, wh1b_ref, wh2_ref, bh2_ref,
               ho_ref, xo_ref):
    g3 = g_ref[...].reshape(BLK, K, TD)
    bsrc3 = g3[:, :, 0:HID]
    xsrc3 = g3[:, :, HID:HID + XW]
    a3 = a_ref[...][:, None, :]
    d23 = d2_ref[...][:, :, None]
    wd3 = wd_ref[...][None, :, :]
    m1 = a3 + bsrc3 + d23 * wd3
    m = _silu(m1).reshape(BLK * K, HID)
    m2 = _silu(jnp.dot(m, we2_ref[...],
                       preferred_element_type=jnp.float32) + be2_ref[...])
    ew = jax.nn.sigmoid(jnp.dot(m2, winf_ref[...],
                                preferred_element_type=jnp.float32)
                        + binf_ref[...])
    w3 = (m2 * ew).reshape(BLK, K, HID)
    agg = jnp.sum(w3, axis=1)
    hin2 = c_ref[...] + jnp.dot(agg, wh1b_ref[...],
                                preferred_element_type=jnp.float32)
    dh = jnp.dot(_silu(hin2), wh2_ref[...],
                 preferred_element_type=jnp.float32) + bh2_ref[...]
    ho_ref[...] = h_ref[...] + dh
    p = _silu(jnp.dot(m2, wx1_ref[...],
                      preferred_element_type=jnp.float32) + bx1_ref[...])
    xmfull = jnp.dot(p, wx2_ref[...], preferred_element_type=jnp.float32)
    xm3 = xmfull.reshape(BLK, K, HID)[:, :, 0:XW]
    xblk = x_ref[...]
    rel3 = xblk[:, None, :] - xsrc3
    dx = jnp.sum(rel3 * xm3, axis=1)
    xo_ref[...] = xblk + dx * mask_ref[...]


def _edge(a, c, h, x, d2, g, mask, wd, we2, be2, winf, binf,
          wx1, bx1, wx2, wh1b, wh2, bh2):
    full = lambda shape: pl.BlockSpec(shape, lambda i: (0, 0))
    return pl.pallas_call(
        _edge_body,
        grid=(NBLK,),
        in_specs=[
            pl.BlockSpec((BLK, HID), lambda i: (i, 0)),
            pl.BlockSpec((BLK, HID), lambda i: (i, 0)),
            pl.BlockSpec((BLK, HID), lambda i: (i, 0)),
            pl.BlockSpec((BLK, XW), lambda i: (i, 0)),
            pl.BlockSpec((BLK, K), lambda i: (i, 0)),
            pl.BlockSpec((BLK * K, TD), lambda i: (i, 0)),
            pl.BlockSpec((BLK, 1), lambda i: (i, 0)),
            full((1, HID)),
            full((HID, HID)),
            full((1, HID)),
            full((HID, HID)),
            full((1, HID)),
            full((HID, HID)),
            full((1, HID)),
            full((HID, HID)),
            full((HID, HID)),
            full((HID, HID)),
            full((1, HID)),
        ],
        out_specs=[
            pl.BlockSpec((BLK, HID), lambda i: (i, 0)),
            pl.BlockSpec((BLK, XW), lambda i: (i, 0)),
        ],
        out_shape=[
            jax.ShapeDtypeStruct((NPAD, HID), jnp.float32),
            jax.ShapeDtypeStruct((NPAD, XW), jnp.float32),
        ],
    )(a, c, h, x, d2, g, mask, wd, we2, be2, winf, binf,
      wx1, bx1, wx2, wh1b, wh2, bh2)


def _head_body(h_ref, wv1_ref, bv1_ref, wv2_ref, bv2_ref, o_ref):
    z = jnp.dot(h_ref[...], wv1_ref[...],
                preferred_element_type=jnp.float32) + bv1_ref[...]
    sp = jnp.maximum(z, 0.0) + jnp.log1p(jnp.exp(-jnp.abs(z)))
    v = sp - jnp.log(2.0)
    o_ref[...] = jnp.dot(v, wv2_ref[...],
                         preferred_element_type=jnp.float32) + bv2_ref[...]


def _head(hl, wv1, bv1, wv2, bv2):
    return pl.pallas_call(
        _head_body,
        out_shape=jax.ShapeDtypeStruct((N_LIG, XW), jnp.float32),
    )(hl, wv1, bv1, wv2, bv2)


def kernel(protein_pos, protein_v, batch_protein, init_ligand_pos,
           init_ligand_v, batch_ligand, params):
    f32 = jnp.float32
    wp = jnp.zeros((32, HID), f32).at[:27, :HID - 1].set(params['W_p'])
    bp = jnp.zeros((1, HID), f32).at[0, :HID - 1].set(params['b_p'])
    wl = jnp.zeros((16, HID), f32).at[:13, :HID - 1].set(params['W_l'])
    bl = jnp.zeros((1, HID), f32).at[0, :HID - 1].set(params['b_l'])
    bl = bl.at[0, HID - 1].set(1.0)
    pv = jnp.zeros((N_PROT, 32), f32).at[:, :27].set(protein_v)
    lv = jnp.zeros((N_LIG, 16), f32).at[:, :13].set(init_ligand_v)
    hp = _embed(pv, wp, bp)
    hl0 = _embed(lv, wl, bl)

    batch_ctx = jnp.concatenate([batch_protein, batch_ligand], axis=0)
    sort_idx = jnp.argsort(batch_ctx)
    batch_all = batch_ctx[sort_idx].astype(jnp.int32)
    is_lig = sort_idx >= N_PROT
    h0 = jnp.concatenate([hp, hl0], axis=0)[sort_idx]
    x0 = jnp.concatenate([protein_pos, init_ligand_pos], axis=0)[sort_idx]

    npad_extra = NPAD - N
    h = jnp.concatenate([h0, jnp.zeros((npad_extra, HID), f32)], axis=0)
    x = jnp.zeros((NPAD, XW), f32).at[:N, 0:3].set(x0)
    bpad = jnp.concatenate(
        [batch_all, jnp.full((npad_extra,), 1 << 20, jnp.int32)])
    brow = bpad.reshape(NPAD, 1)
    bcol = bpad.reshape(1, NPAD)
    maskpad = jnp.concatenate(
        [is_lig.astype(f32), jnp.zeros((npad_extra,), f32)]).reshape(NPAD, 1)

    for lp in params['layers']:
        we1a = lp['We1'][0:HID]
        we1b = lp['We1'][HID:2 * HID]
        wd = lp['We1'][2 * HID:2 * HID + 1]
        be1 = lp['be1'].reshape(1, HID)
        wh1a = lp['Wh1'][0:HID]
        wh1b = lp['Wh1'][HID:2 * HID]
        bh1 = lp['bh1'].reshape(1, HID)
        winf = jnp.broadcast_to(lp['Winf'], (HID, HID))
        binf = jnp.broadcast_to(lp['binf'].reshape(1, 1), (1, HID))
        wx2 = jnp.broadcast_to(lp['Wx2'], (HID, HID))

        xt = jnp.zeros((8, NPAD), f32).at[0:3, :].set(x[:, 0:3].T)
        src, d2 = _knn(x, xt, brow, bcol)
        a, c, t = _prep(h, x, we1a, we1b, wh1a, be1, bh1)
        g = _sc_gather(t, src.reshape(E))
        h, x = _edge(a, c, h, x, d2, g, maskpad,
                     wd, lp['We2'], lp['be2'].reshape(1, HID),
                     winf, binf, lp['Wx1'], lp['bx1'].reshape(1, HID),
                     wx2, wh1b, lp['Wh2'], lp['bh2'].reshape(1, HID))

    h_final = h[:N]
    lig_idx = jnp.nonzero(is_lig, size=N_LIG)[0]
    final_ligand_h = h_final[lig_idx]
    final_ligand_pos = x[:N][lig_idx][:, 0:3]

    wv2 = jnp.zeros((HID, XW), f32).at[:, :13].set(params['Wv2'])
    bv2 = jnp.zeros((1, XW), f32).at[0, :13].set(params['bv2'])
    v16 = _head(final_ligand_h, params['Wv1'], params['bv1'].reshape(1, HID),
                wv2, bv2)
    final_ligand_v = v16[:, :13]
    return final_ligand_pos, final_ligand_v, h_final, final_ligand_h


# speedup vs baseline: 1.2340x; 1.2340x over previous
import functools
from typing import Any

import jax
import jax.numpy as jnp
from jax import lax
from jax.experimental import pallas as pl
from jax.experimental.pallas import tpu as pltpu
from jax.experimental.pallas import tpu_sc as plsc

HID = 128
K = 32
N_PROT = 4800
N_LIG = 1200
N = N_PROT + N_LIG
NPAD = 6144
XW = 16
TD = 256
BLK = 256
NBLK = NPAD // BLK
E = NPAD * K
CHUNK = 128


def _silu(v):
    return v * jax.nn.sigmoid(v)


def _embed_body(v_ref, w_ref, b_ref, o_ref):
    o_ref[...] = jnp.dot(v_ref[...], w_ref[...],
                         preferred_element_type=jnp.float32) + b_ref[...]


def _embed(v, w, b):
    return pl.pallas_call(
        _embed_body,
        out_shape=jax.ShapeDtypeStruct((v.shape[0], HID), jnp.float32),
    )(v, w, b)


def _knn_body(xrow_ref, xt_ref, brow_ref, bcol_ref, src_ref, d2_ref):
    i = pl.program_id(0)
    xi = xrow_ref[...]
    bi = brow_ref[...]
    bj = bcol_ref[...]
    d2 = jnp.zeros((BLK, NPAD), jnp.float32)
    for c in range(3):
        diff = xi[:, c:c + 1] - xt_ref[c:c + 1, :]
        d2 = d2 + diff * diff
    rowid = i * BLK + lax.broadcasted_iota(jnp.int32, (BLK, 1), 0)
    colid = lax.broadcasted_iota(jnp.int32, (BLK, NPAD), 1)
    same = (bi == bj) & (rowid != colid)
    d2m = jnp.where(same, d2, jnp.float32(1e30))
    worst = jnp.float32(0.0)
    for k in range(K):
        mv = jnp.min(d2m, axis=1, keepdims=True)
        idx = jnp.min(jnp.where(d2m == mv, colid, jnp.int32(2**30)),
                      axis=1, keepdims=True)
        src_ref[:, k:k + 1] = idx
        d2_ref[:, k:k + 1] = mv
        worst = jnp.maximum(worst, jnp.max(mv))
        d2m = jnp.where(colid == idx, jnp.float32(2e30), d2m)

    @pl.when(worst >= jnp.float32(1e30))
    def _fixup():
        for k in range(K):
            idx = src_ref[:, k:k + 1]
            val = jnp.min(jnp.where(colid == idx, d2, jnp.float32(3e30)),
                          axis=1, keepdims=True)
            d2_ref[:, k:k + 1] = val


def _knn(xrow, xt, brow, bcol):
    return pl.pallas_call(
        _knn_body,
        grid=(NBLK,),
        in_specs=[
            pl.BlockSpec((BLK, XW), lambda i: (i, 0)),
            pl.BlockSpec((8, NPAD), lambda i: (0, 0)),
            pl.BlockSpec((BLK, 1), lambda i: (i, 0)),
            pl.BlockSpec((1, NPAD), lambda i: (0, 0)),
        ],
        out_specs=[
            pl.BlockSpec((BLK, K), lambda i: (i, 0)),
            pl.BlockSpec((BLK, K), lambda i: (i, 0)),
        ],
        out_shape=[
            jax.ShapeDtypeStruct((NPAD, K), jnp.int32),
            jax.ShapeDtypeStruct((NPAD, K), jnp.float32),
        ],
    )(xrow, xt, brow, bcol)


def _prep_body(h_ref, x_ref, we1a_ref, we1b_ref, wh1a_ref, be1_ref, bh1_ref,
               a_ref, c_ref, t_ref):
    h = h_ref[...]
    a_ref[...] = jnp.dot(h, we1a_ref[...],
                         preferred_element_type=jnp.float32) + be1_ref[...]
    c_ref[...] = jnp.dot(h, wh1a_ref[...],
                         preferred_element_type=jnp.float32) + bh1_ref[...]
    b = jnp.dot(h, we1b_ref[...], preferred_element_type=jnp.float32)
    t_ref[...] = jnp.concatenate(
        [b, x_ref[...], jnp.zeros((BLK, TD - HID - XW), jnp.float32)], axis=1)


def _prep(h, x, we1a, we1b, wh1a, be1, bh1):
    return pl.pallas_call(
        _prep_body,
        grid=(NBLK,),
        in_specs=[
            pl.BlockSpec((BLK, HID), lambda i: (i, 0)),
            pl.BlockSpec((BLK, XW), lambda i: (i, 0)),
            pl.BlockSpec((HID, HID), lambda i: (0, 0)),
            pl.BlockSpec((HID, HID), lambda i: (0, 0)),
            pl.BlockSpec((HID, HID), lambda i: (0, 0)),
            pl.BlockSpec((1, HID), lambda i: (0, 0)),
            pl.BlockSpec((1, HID), lambda i: (0, 0)),
        ],
        out_specs=[
            pl.BlockSpec((BLK, HID), lambda i: (i, 0)),
            pl.BlockSpec((BLK, HID), lambda i: (i, 0)),
            pl.BlockSpec((BLK, TD), lambda i: (i, 0)),
        ],
        out_shape=[
            jax.ShapeDtypeStruct((NPAD, HID), jnp.float32),
            jax.ShapeDtypeStruct((NPAD, HID), jnp.float32),
            jax.ShapeDtypeStruct((NPAD, TD), jnp.float32),
        ],
    )(h, x, we1a, we1b, wh1a, be1, bh1)


_PER_W = E // 32
_NCHUNK = _PER_W // CHUNK


def _sc_gather(table, idx):
    mesh = plsc.VectorSubcoreMesh(core_axis_name="c", subcore_axis_name="s")

    @functools.partial(
        pl.kernel,
        mesh=mesh,
        out_type=jax.ShapeDtypeStruct((E, TD), jnp.float32),
        scratch_types=[
            pltpu.VMEM((_PER_W,), jnp.int32),
            pltpu.VMEM((CHUNK, TD), jnp.float32),
            pltpu.VMEM((CHUNK, TD), jnp.float32),
            pltpu.SemaphoreType.DMA,
            pltpu.SemaphoreType.DMA,
        ],
    )
    def k(table_hbm, idx_hbm, out_hbm, idx_v, rows0, rows1, sem0, sem1):
        wid = lax.axis_index("s") * 2 + lax.axis_index("c")
        base0 = wid * _PER_W
        pltpu.sync_copy(idx_hbm.at[pl.ds(base0, _PER_W)], idx_v)
        bufs = [(rows0, sem0), (rows1, sem1)]

        def fire(c):
            r, s = bufs[c % 2]
            return pltpu.async_copy(
                table_hbm.at[idx_v.at[pl.ds(c * CHUNK, CHUNK)]], r, s)

        cp = fire(0)
        for c in range(_NCHUNK):
            nxt = fire(c + 1) if c + 1 < _NCHUNK else None
            cp.wait()
            r, _ = bufs[c % 2]
            pltpu.sync_copy(r, out_hbm.at[pl.ds(base0 + c * CHUNK, CHUNK)])
            cp = nxt

    return k(table, idx)


def _edge_body(a_ref, c_ref, h_ref, x_ref, d2_ref, g_ref, mask_ref,
               wd_ref, we2_ref, be2_ref, winf_ref, binf_ref,
               wx1_ref, bx1_ref, wx2_ref, wh1b_ref, wh2_ref, bh2_ref,
               ho_ref, xo_ref):
    g3 = g_ref[...].reshape(BLK, K, TD)
    bsrc3 = g3[:, :, 0:HID]
    xsrc3 = g3[:, :, HID:HID + XW]
    a3 = a_ref[...][:, None, :]
    d23 = d2_ref[...][:, :, None]
    wd3 = wd_ref[...][None, :, :]
    m1 = a3 + bsrc3 + d23 * wd3
    m = _silu(m1).reshape(BLK * K, HID)
    m2 = _silu(jnp.dot(m, we2_ref[...],
                       preferred_element_type=jnp.float32) + be2_ref[...])
    ew = jax.nn.sigmoid(jnp.dot(m2, winf_ref[...],
                                preferred_element_type=jnp.float32)
                        + binf_ref[...])
    w3 = (m2 * ew).reshape(BLK, K, HID)
    agg = jnp.sum(w3, axis=1)
    hin2 = c_ref[...] + jnp.dot(agg, wh1b_ref[...],
                                preferred_element_type=jnp.float32)
    dh = jnp.dot(_silu(hin2), wh2_ref[...],
                 preferred_element_type=jnp.float32) + bh2_ref[...]
    ho_ref[...] = h_ref[...] + dh
    p = _silu(jnp.dot(m2, wx1_ref[...],
                      preferred_element_type=jnp.float32) + bx1_ref[...])
    xmfull = jnp.dot(p, wx2_ref[...], preferred_element_type=jnp.float32)
    xm3 = xmfull.reshape(BLK, K, HID)[:, :, 0:XW]
    xblk = x_ref[...]
    rel3 = xblk[:, None, :] - xsrc3
    dx = jnp.sum(rel3 * xm3, axis=1)
    xo_ref[...] = xblk + dx * mask_ref[...]


def _edge(a, c, h, x, d2, g, mask, wd, we2, be2, winf, binf,
          wx1, bx1, wx2, wh1b, wh2, bh2):
    full = lambda shape: pl.BlockSpec(shape, lambda i: (0, 0))
    return pl.pallas_call(
        _edge_body,
        grid=(NBLK,),
        in_specs=[
            pl.BlockSpec((BLK, HID), lambda i: (i, 0)),
            pl.BlockSpec((BLK, HID), lambda i: (i, 0)),
            pl.BlockSpec((BLK, HID), lambda i: (i, 0)),
            pl.BlockSpec((BLK, XW), lambda i: (i, 0)),
            pl.BlockSpec((BLK, K), lambda i: (i, 0)),
            pl.BlockSpec((BLK * K, TD), lambda i: (i, 0)),
            pl.BlockSpec((BLK, 1), lambda i: (i, 0)),
            full((1, HID)),
            full((HID, HID)),
            full((1, HID)),
            full((HID, HID)),
            full((1, HID)),
            full((HID, HID)),
            full((1, HID)),
            full((HID, HID)),
            full((HID, HID)),
            full((HID, HID)),
            full((1, HID)),
        ],
        out_specs=[
            pl.BlockSpec((BLK, HID), lambda i: (i, 0)),
            pl.BlockSpec((BLK, XW), lambda i: (i, 0)),
        ],
        out_shape=[
            jax.ShapeDtypeStruct((NPAD, HID), jnp.float32),
            jax.ShapeDtypeStruct((NPAD, XW), jnp.float32),
        ],
    )(a, c, h, x, d2, g, mask, wd, we2, be2, winf, binf,
      wx1, bx1, wx2, wh1b, wh2, bh2)


def _head_body(h_ref, wv1_ref, bv1_ref, wv2_ref, bv2_ref, o_ref):
    z = jnp.dot(h_ref[...], wv1_ref[...],
                preferred_element_type=jnp.float32) + bv1_ref[...]
    sp = jnp.maximum(z, 0.0) + jnp.log1p(jnp.exp(-jnp.abs(z)))
    v = sp - jnp.log(2.0)
    o_ref[...] = jnp.dot(v, wv2_ref[...],
                         preferred_element_type=jnp.float32) + bv2_ref[...]


def _head(hl, wv1, bv1, wv2, bv2):
    return pl.pallas_call(
        _head_body,
        out_shape=jax.ShapeDtypeStruct((N_LIG, XW), jnp.float32),
    )(hl, wv1, bv1, wv2, bv2)


def kernel(protein_pos, protein_v, batch_protein, init_ligand_pos,
           init_ligand_v, batch_ligand, params):
    f32 = jnp.float32
    wp = jnp.zeros((32, HID), f32).at[:27, :HID - 1].set(params['W_p'])
    bp = jnp.zeros((1, HID), f32).at[0, :HID - 1].set(params['b_p'])
    wl = jnp.zeros((16, HID), f32).at[:13, :HID - 1].set(params['W_l'])
    bl = jnp.zeros((1, HID), f32).at[0, :HID - 1].set(params['b_l'])
    bl = bl.at[0, HID - 1].set(1.0)
    pv = jnp.zeros((N_PROT, 32), f32).at[:, :27].set(protein_v)
    lv = jnp.zeros((N_LIG, 16), f32).at[:, :13].set(init_ligand_v)
    hp = _embed(pv, wp, bp)
    hl0 = _embed(lv, wl, bl)

    batch_ctx = jnp.concatenate([batch_protein, batch_ligand], axis=0)
    sort_idx = jnp.argsort(batch_ctx)
    batch_all = batch_ctx[sort_idx].astype(jnp.int32)
    is_lig = sort_idx >= N_PROT
    h0 = jnp.concatenate([hp, hl0], axis=0)[sort_idx]
    x0 = jnp.concatenate([protein_pos, init_ligand_pos], axis=0)[sort_idx]

    npad_extra = NPAD - N
    h = jnp.concatenate([h0, jnp.zeros((npad_extra, HID), f32)], axis=0)
    x = jnp.zeros((NPAD, XW), f32).at[:N, 0:3].set(x0)
    bpad = jnp.concatenate(
        [batch_all, jnp.full((npad_extra,), 1 << 20, jnp.int32)])
    brow = bpad.reshape(NPAD, 1)
    bcol = bpad.reshape(1, NPAD)
    maskpad = jnp.concatenate(
        [is_lig.astype(f32), jnp.zeros((npad_extra,), f32)]).reshape(NPAD, 1)

    for lp in params['layers']:
        we1a = lp['We1'][0:HID]
        we1b = lp['We1'][HID:2 * HID]
        wd = lp['We1'][2 * HID:2 * HID + 1]
        be1 = lp['be1'].reshape(1, HID)
        wh1a = lp['Wh1'][0:HID]
        wh1b = lp['Wh1'][HID:2 * HID]
        bh1 = lp['bh1'].reshape(1, HID)
        winf = jnp.broadcast_to(lp['Winf'], (HID, HID))
        binf = jnp.broadcast_to(lp['binf'].reshape(1, 1), (1, HID))
        wx2 = jnp.broadcast_to(lp['Wx2'], (HID, HID))

        xt = jnp.zeros((8, NPAD), f32).at[0:3, :].set(x[:, 0:3].T)
        src, d2 = _knn(x, xt, brow, bcol)
        a, c, t = _prep(h, x, we1a, we1b, wh1a, be1, bh1)
        g = _sc_gather(t, src.reshape(E))
        h, x = _edge(a, c, h, x, d2, g, maskpad,
                     wd, lp['We2'], lp['be2'].reshape(1, HID),
                     winf, binf, lp['Wx1'], lp['bx1'].reshape(1, HID),
                     wx2, wh1b, lp['Wh2'], lp['bh2'].reshape(1, HID))

    h_final = h[:N]
    lig_idx = jnp.nonzero(is_lig, size=N_LIG)[0]
    final_ligand_h = h_final[lig_idx]
    final_ligand_pos = x[:N][lig_idx][:, 0:3]

    wv2 = jnp.zeros((HID, XW), f32).at[:, :13].set(params['Wv2'])
    bv2 = jnp.zeros((1, XW), f32).at[0, :13].set(params['bv2'])
    v16 = _head(final_ligand_h, params['Wv1'], params['bv1'].reshape(1, HID),
                wv2, bv2)
    final_ligand_v = v16[:, :13]
    return final_ligand_pos, final_ligand_v, h_final, final_ligand_h


# speedup vs baseline: 1.2342x; 1.0002x over previous
import functools

import jax
import jax.numpy as jnp
from jax import lax
from jax.experimental import pallas as pl
from jax.experimental.pallas import tpu as pltpu
from jax.experimental.pallas import tpu_sc as plsc

HID = 128
K = 32
N_PROT = 4800
N_LIG = 1200
N = N_PROT + N_LIG
NPAD = 6144
XW = 16
TD = 256
BLK = 256
NBLK = NPAD // BLK
E = NPAD * K
CHUNK = 128


def _silu(v):
    return v * jax.nn.sigmoid(v)


def _embed_body(v_ref, w_ref, b_ref, o_ref):
    o_ref[...] = jnp.dot(v_ref[...], w_ref[...],
                         preferred_element_type=jnp.float32) + b_ref[...]


def _embed(v, w, b):
    return pl.pallas_call(
        _embed_body,
        out_shape=jax.ShapeDtypeStruct((v.shape[0], HID), jnp.float32),
    )(v, w, b)


def _knn_body(xrow_ref, xt_ref, brow_ref, bcol_ref, src_ref, d2_ref):
    i = pl.program_id(0)
    xi = xrow_ref[...]
    bi = brow_ref[...]
    bj = bcol_ref[...]
    d2 = jnp.zeros((BLK, NPAD), jnp.float32)
    for c in range(3):
        diff = xi[:, c:c + 1] - xt_ref[c:c + 1, :]
        d2 = d2 + diff * diff
    rowid = i * BLK + lax.broadcasted_iota(jnp.int32, (BLK, 1), 0)
    colid = lax.broadcasted_iota(jnp.int32, (BLK, NPAD), 1)
    same = (bi == bj) & (rowid != colid)
    d2m = jnp.where(same, d2, jnp.float32(1e30))
    worst = jnp.float32(0.0)
    for k in range(K):
        mv = jnp.min(d2m, axis=1, keepdims=True)
        idx = jnp.min(jnp.where(d2m == mv, colid, jnp.int32(2**30)),
                      axis=1, keepdims=True)
        src_ref[:, k:k + 1] = idx
        d2_ref[:, k:k + 1] = mv
        worst = jnp.maximum(worst, jnp.max(mv))
        d2m = jnp.where(colid == idx, jnp.float32(2e30), d2m)

    @pl.when(worst >= jnp.float32(1e30))
    def _fixup():
        for k in range(K):
            idx = src_ref[:, k:k + 1]
            val = jnp.min(jnp.where(colid == idx, d2, jnp.float32(3e30)),
                          axis=1, keepdims=True)
            d2_ref[:, k:k + 1] = val


def _knn(xrow, xt, brow, bcol):
    return pl.pallas_call(
        _knn_body,
        grid=(NBLK,),
        in_specs=[
            pl.BlockSpec((BLK, XW), lambda i: (i, 0)),
            pl.BlockSpec((8, NPAD), lambda i: (0, 0)),
            pl.BlockSpec((BLK, 1), lambda i: (i, 0)),
            pl.BlockSpec((1, NPAD), lambda i: (0, 0)),
        ],
        out_specs=[
            pl.BlockSpec((BLK, K), lambda i: (i, 0)),
            pl.BlockSpec((BLK, K), lambda i: (i, 0)),
        ],
        out_shape=[
            jax.ShapeDtypeStruct((NPAD, K), jnp.int32),
            jax.ShapeDtypeStruct((NPAD, K), jnp.float32),
        ],
    )(xrow, xt, brow, bcol)


def _prep_body(h_ref, x_ref, we1a_ref, we1b_ref, wh1a_ref, be1_ref, bh1_ref,
               a_ref, c_ref, t_ref):
    h = h_ref[...]
    a_ref[...] = jnp.dot(h, we1a_ref[...],
                         preferred_element_type=jnp.float32) + be1_ref[...]
    c_ref[...] = jnp.dot(h, wh1a_ref[...],
                         preferred_element_type=jnp.float32) + bh1_ref[...]
    b = jnp.dot(h, we1b_ref[...], preferred_element_type=jnp.float32)
    t_ref[...] = jnp.concatenate(
        [b, x_ref[...], jnp.zeros((BLK, TD - HID - XW), jnp.float32)], axis=1)


def _prep(h, x, we1a, we1b, wh1a, be1, bh1):
    return pl.pallas_call(
        _prep_body,
        grid=(NBLK,),
        in_specs=[
            pl.BlockSpec((BLK, HID), lambda i: (i, 0)),
            pl.BlockSpec((BLK, XW), lambda i: (i, 0)),
            pl.BlockSpec((HID, HID), lambda i: (0, 0)),
            pl.BlockSpec((HID, HID), lambda i: (0, 0)),
            pl.BlockSpec((HID, HID), lambda i: (0, 0)),
            pl.BlockSpec((1, HID), lambda i: (0, 0)),
            pl.BlockSpec((1, HID), lambda i: (0, 0)),
        ],
        out_specs=[
            pl.BlockSpec((BLK, HID), lambda i: (i, 0)),
            pl.BlockSpec((BLK, HID), lambda i: (i, 0)),
            pl.BlockSpec((BLK, TD), lambda i: (i, 0)),
        ],
        out_shape=[
            jax.ShapeDtypeStruct((NPAD, HID), jnp.float32),
            jax.ShapeDtypeStruct((NPAD, HID), jnp.float32),
            jax.ShapeDtypeStruct((NPAD, TD), jnp.float32),
        ],
    )(h, x, we1a, we1b, wh1a, be1, bh1)


_PER_W = E // 32
_NCHUNK = _PER_W // CHUNK


def _sc_gather(table, idx):
    mesh = plsc.VectorSubcoreMesh(core_axis_name="c", subcore_axis_name="s")

    @functools.partial(
        pl.kernel,
        mesh=mesh,
        out_type=jax.ShapeDtypeStruct((E, TD), jnp.float32),
        scratch_types=[
            pltpu.VMEM((_PER_W,), jnp.int32),
            pltpu.VMEM((CHUNK, TD), jnp.float32),
            pltpu.VMEM((CHUNK, TD), jnp.float32),
            pltpu.SemaphoreType.DMA,
            pltpu.SemaphoreType.DMA,
        ],
    )
    def k(table_hbm, idx_hbm, out_hbm, idx_v, rows0, rows1, sem0, sem1):
        wid = lax.axis_index("s") * 2 + lax.axis_index("c")
        base0 = wid * _PER_W
        pltpu.sync_copy(idx_hbm.at[pl.ds(base0, _PER_W)], idx_v)
        bufs = [(rows0, sem0), (rows1, sem1)]

        def fire(c):
            r, s = bufs[c % 2]
            return pltpu.async_copy(
                table_hbm.at[idx_v.at[pl.ds(c * CHUNK, CHUNK)]], r, s)

        cp = fire(0)
        for c in range(_NCHUNK):
            nxt = fire(c + 1) if c + 1 < _NCHUNK else None
            cp.wait()
            r, _ = bufs[c % 2]
            pltpu.sync_copy(r, out_hbm.at[pl.ds(base0 + c * CHUNK, CHUNK)])
            cp = nxt

    return k(table, idx)


def _edge_body(a_ref, c_ref, h_ref, x_ref, d2_ref, g_ref, mask_ref,
               wd_ref, we2_ref, be2_ref, winf_ref, binf_ref,
               wx1_ref, bx1_ref, wx2_ref, wh1b_ref, wh2_ref, bh2_ref,
               ho_ref, xo_ref):
    g3 = g_ref[...].reshape(BLK, K, TD)
    bsrc3 = g3[:, :, 0:HID]
    xsrc3 = g3[:, :, HID:HID + XW]
    a3 = a_ref[...][:, None, :]
    d23 = d2_ref[...][:, :, None]
    wd3 = wd_ref[...][None, :, :]
    m1 = a3 + bsrc3 + d23 * wd3
    m = _silu(m1).reshape(BLK * K, HID)
    m2 = _silu(jnp.dot(m, we2_ref[...],
                       preferred_element_type=jnp.float32) + be2_ref[...])
    ew = jax.nn.sigmoid(jnp.dot(m2, winf_ref[...],
                                preferred_element_type=jnp.float32)
                        + binf_ref[...])
    w3 = (m2 * ew).reshape(BLK, K, HID)
    agg = jnp.sum(w3, axis=1)
    hin2 = c_ref[...] + jnp.dot(agg, wh1b_ref[...],
                                preferred_element_type=jnp.float32)
    dh = jnp.dot(_silu(hin2), wh2_ref[...],
                 preferred_element_type=jnp.float32) + bh2_ref[...]
    ho_ref[...] = h_ref[...] + dh
    p = _silu(jnp.dot(m2, wx1_ref[...],
                      preferred_element_type=jnp.float32) + bx1_ref[...])
    xmfull = jnp.dot(p, wx2_ref[...], preferred_element_type=jnp.float32)
    xm3 = xmfull.reshape(BLK, K, HID)[:, :, 0:XW]
    xblk = x_ref[...]
    rel3 = xblk[:, None, :] - xsrc3
    dx = jnp.sum(rel3 * xm3, axis=1)
    xo_ref[...] = xblk + dx * mask_ref[...]


def _edge(a, c, h, x, d2, g, mask, wd, we2, be2, winf, binf,
          wx1, bx1, wx2, wh1b, wh2, bh2):
    full = lambda shape: pl.BlockSpec(shape, lambda i: (0, 0))
    return pl.pallas_call(
        _edge_body,
        grid=(NBLK,),
        in_specs=[
            pl.BlockSpec((BLK, HID), lambda i: (i, 0)),
            pl.BlockSpec((BLK, HID), lambda i: (i, 0)),
            pl.BlockSpec((BLK, HID), lambda i: (i, 0)),
            pl.BlockSpec((BLK, XW), lambda i: (i, 0)),
            pl.BlockSpec((BLK, K), lambda i: (i, 0)),
            pl.BlockSpec((BLK * K, TD), lambda i: (i, 0)),
            pl.BlockSpec((BLK, 1), lambda i: (i, 0)),
            full((1, HID)),
            full((HID, HID)),
            full((1, HID)),
            full((HID, HID)),
            full((1, HID)),
            full((HID, HID)),
            full((1, HID)),
            full((HID, HID)),
            full((HID, HID)),
            full((HID, HID)),
            full((1, HID)),
        ],
        out_specs=[
            pl.BlockSpec((BLK, HID), lambda i: (i, 0)),
            pl.BlockSpec((BLK, XW), lambda i: (i, 0)),
        ],
        out_shape=[
            jax.ShapeDtypeStruct((NPAD, HID), jnp.float32),
            jax.ShapeDtypeStruct((NPAD, XW), jnp.float32),
        ],
    )(a, c, h, x, d2, g, mask, wd, we2, be2, winf, binf,
      wx1, bx1, wx2, wh1b, wh2, bh2)


def _head_body(h_ref, wv1_ref, bv1_ref, wv2_ref, bv2_ref, o_ref):
    z = jnp.dot(h_ref[...], wv1_ref[...],
                preferred_element_type=jnp.float32) + bv1_ref[...]
    sp = jnp.maximum(z, 0.0) + jnp.log1p(jnp.exp(-jnp.abs(z)))
    v = sp - jnp.log(2.0)
    o_ref[...] = jnp.dot(v, wv2_ref[...],
                         preferred_element_type=jnp.float32) + bv2_ref[...]


def _head(hl, wv1, bv1, wv2, bv2):
    return pl.pallas_call(
        _head_body,
        out_shape=jax.ShapeDtypeStruct((N_LIG, XW), jnp.float32),
    )(hl, wv1, bv1, wv2, bv2)


def kernel(protein_pos, protein_v, batch_protein, init_ligand_pos,
           init_ligand_v, batch_ligand, params):
    f32 = jnp.float32
    wp = jnp.zeros((32, HID), f32).at[:27, :HID - 1].set(params['W_p'])
    bp = jnp.zeros((1, HID), f32).at[0, :HID - 1].set(params['b_p'])
    wl = jnp.zeros((16, HID), f32).at[:13, :HID - 1].set(params['W_l'])
    bl = jnp.zeros((1, HID), f32).at[0, :HID - 1].set(params['b_l'])
    bl = bl.at[0, HID - 1].set(1.0)
    pv = jnp.zeros((N_PROT, 32), f32).at[:, :27].set(protein_v)
    lv = jnp.zeros((N_LIG, 16), f32).at[:, :13].set(init_ligand_v)
    hp = _embed(pv, wp, bp)
    hl0 = _embed(lv, wl, bl)

    batch_ctx = jnp.concatenate([batch_protein, batch_ligand], axis=0)
    sort_idx = jnp.argsort(batch_ctx)
    batch_all = batch_ctx[sort_idx].astype(jnp.int32)
    is_lig = sort_idx >= N_PROT
    h0 = jnp.concatenate([hp, hl0], axis=0)[sort_idx]
    x0 = jnp.concatenate([protein_pos, init_ligand_pos], axis=0)[sort_idx]

    npad_extra = NPAD - N
    h = jnp.concatenate([h0, jnp.zeros((npad_extra, HID), f32)], axis=0)
    x = jnp.zeros((NPAD, XW), f32).at[:N, 0:3].set(x0)
    bpad = jnp.concatenate(
        [batch_all, jnp.full((npad_extra,), 1 << 20, jnp.int32)])
    brow = bpad.reshape(NPAD, 1)
    bcol = bpad.reshape(1, NPAD)
    maskpad = jnp.concatenate(
        [is_lig.astype(f32), jnp.zeros((npad_extra,), f32)]).reshape(NPAD, 1)

    for lp in params['layers']:
        we1a = lp['We1'][0:HID]
        we1b = lp['We1'][HID:2 * HID]
        wd = lp['We1'][2 * HID:2 * HID + 1]
        be1 = lp['be1'].reshape(1, HID)
        wh1a = lp['Wh1'][0:HID]
        wh1b = lp['Wh1'][HID:2 * HID]
        bh1 = lp['bh1'].reshape(1, HID)
        winf = jnp.broadcast_to(lp['Winf'], (HID, HID))
        binf = jnp.broadcast_to(lp['binf'].reshape(1, 1), (1, HID))
        wx2 = jnp.broadcast_to(lp['Wx2'], (HID, HID))

        xt = jnp.zeros((8, NPAD), f32).at[0:3, :].set(x[:, 0:3].T)
        src, d2 = _knn(x, xt, brow, bcol)
        a, c, t = _prep(h, x, we1a, we1b, wh1a, be1, bh1)
        g = _sc_gather(t, src.reshape(E))
        h, x = _edge(a, c, h, x, d2, g, maskpad,
                     wd, lp['We2'], lp['be2'].reshape(1, HID),
                     winf, binf, lp['Wx1'], lp['bx1'].reshape(1, HID),
                     wx2, wh1b, lp['Wh2'], lp['bh2'].reshape(1, HID))

    h_final = h[:N]
    lig_idx = jnp.nonzero(is_lig, size=N_LIG)[0]
    final_ligand_h = h_final[lig_idx]
    final_ligand_pos = x[:N][lig_idx][:, 0:3]

    wv2 = jnp.zeros((HID, XW), f32).at[:, :13].set(params['Wv2'])
    bv2 = jnp.zeros((1, XW), f32).at[0, :13].set(params['bv2'])
    v16 = _head(final_ligand_h, params['Wv1'], params['bv1'].reshape(1, HID),
                wv2, bv2)
    final_ligand_v = v16[:, :13]
    return final_ligand_pos, final_ligand_v, h_final, final_ligand_h
